# Initial kernel scaffold; baseline (speedup 1.0000x reference)
import jax
import jax.numpy as jnp
from jax.experimental import pallas as pl


def kernel(x, edge_index, edge_attr, batch, We1, be1, W1a, b1a, W1b, b1b, We2, be2, W2a, b2a, W2b, b2b):
    raise NotImplementedError("write your pallas kernel here")


# speedup vs baseline: 3.1269x; 3.1269x over previous
import functools

import jax
import jax.numpy as jnp
from jax import lax
from jax.experimental import pallas as pl
from jax.experimental.pallas import tpu as pltpu
from jax.experimental.pallas import tpu_sc as plsc

N_GRAPHS = 64

NC = 2
NS = 16
NW = NC * NS

SUB = 125
ROWS_PER_CHUNK = 8
C = ROWS_PER_CHUNK * SUB

W = 64

AGG_N = 10240
ROWS_PT = AGG_N // NS


def _sc_edge_stage_body(n_chunks, x_hbm, src_hbm, dst_hbm, e_hbm,
                        out_hbm, buf, sidx, didx, agg, sem):
    c = lax.axis_index("c")
    s = lax.axis_index("s")
    wid = c * NS + s
    base = s * ROWS_PT

    def _zrow(r, _):
        for j in range(W // 16):
            buf[r, pl.ds(j * 16, 16)] = jnp.zeros((16,), jnp.float32)
        return 0

    lax.fori_loop(0, C, _zrow, 0)
    pltpu.sync_copy(buf.at[pl.ds(0, ROWS_PT)], agg.at[pl.ds(base, ROWS_PT)])
    plsc.subcore_barrier()

    def _chunk(k, _):
        eb = (wid * n_chunks + k) * C
        pltpu.sync_copy(src_hbm.at[wid, pl.ds(k * ROWS_PER_CHUNK,
                                              ROWS_PER_CHUNK)], sidx)
        pltpu.sync_copy(dst_hbm.at[wid, pl.ds(k * ROWS_PER_CHUNK,
                                              ROWS_PER_CHUNK)], didx)
        pltpu.sync_copy(e_hbm.at[pl.ds(eb, C)], buf)
        descs = [
            pltpu.async_copy(x_hbm.at[sidx.at[j]],
                             buf.at[pl.ds(j * SUB, SUB)], sem, add=True)
            for j in range(ROWS_PER_CHUNK)
        ]
        for d in descs:
            d.wait()

        def _rrow(r, _):
            for j in range(W // 16):
                v = buf[r, pl.ds(j * 16, 16)]
                buf[r, pl.ds(j * 16, 16)] = jnp.maximum(v, 0.0)
            return 0

        lax.fori_loop(0, C, _rrow, 0)
        for j in range(ROWS_PER_CHUNK):
            pltpu.sync_copy(buf.at[pl.ds(j * SUB, SUB)],
                            agg.at[didx.at[j]], add=True)
        return 0

    lax.fori_loop(0, n_chunks, _chunk, 0)
    plsc.subcore_barrier()
    pltpu.sync_copy(agg.at[pl.ds(base, ROWS_PT)],
                    out_hbm.at[c, pl.ds(base, ROWS_PT)])


def _sc_edge_stage(x, src3d, dst3d, e):
    E = e.shape[0]
    n_chunks = E // (NW * C)
    assert E == NW * C * n_chunks
    mesh = plsc.VectorSubcoreMesh(core_axis_name="c", subcore_axis_name="s")
    fn = pl.kernel(
        functools.partial(_sc_edge_stage_body, n_chunks),
        out_type=jax.ShapeDtypeStruct((NC, AGG_N, W), jnp.float32),
        mesh=mesh,
        compiler_params=pltpu.CompilerParams(use_tc_tiling_on_sc=False),
        scratch_types=[
            pltpu.VMEM((C, W), jnp.float32),
            pltpu.VMEM((ROWS_PER_CHUNK, SUB), jnp.int32),
            pltpu.VMEM((ROWS_PER_CHUNK, SUB), jnp.int32),
            pltpu.VMEM_SHARED((AGG_N, W), jnp.float32),
            pltpu.SemaphoreType.DMA,
        ],
    )
    return fn(x, src3d, dst3d, e)


def _edge_lin_kernel(ea_ref, w1l_ref, b1l_ref, w1r_ref, b1r_ref, w2_ref,
                     b2_ref, o1l_ref, o1r_ref, o2_ref):
    ea = ea_ref[...]
    o1l_ref[...] = jnp.dot(ea, w1l_ref[...],
                           preferred_element_type=jnp.float32) + b1l_ref[...]
    o1r_ref[...] = jnp.dot(ea, w1r_ref[...],
                           preferred_element_type=jnp.float32) + b1r_ref[...]
    o2_ref[...] = jnp.dot(ea, w2_ref[...],
                          preferred_element_type=jnp.float32) + b2_ref[...]


def _edge_linears(edge_attr, W1l, b1l, W1r, b1r, We2, be2):
    E, DE = edge_attr.shape
    BE = 2000
    grid = E // BE
    wspec = pl.BlockSpec((DE, W), lambda i: (0, 0))
    bspec = pl.BlockSpec((1, W), lambda i: (0, 0))
    ospec = pl.BlockSpec((BE, W), lambda i: (i, 0))
    oshape = jax.ShapeDtypeStruct((E, W), jnp.float32)
    return pl.pallas_call(
        _edge_lin_kernel,
        grid=(grid,),
        in_specs=[pl.BlockSpec((BE, DE), lambda i: (i, 0)),
                  wspec, bspec, wspec, bspec, wspec, bspec],
        out_specs=[ospec, ospec, ospec],
        out_shape=[oshape, oshape, oshape],
    )(edge_attr, W1l, b1l, W1r, b1r, We2, be2)


def _mlp1_kernel(x_ref, aggl_ref, aggr_ref, w1a_ref, b1a_ref, w1b_ref,
                 b1b_ref, o_ref):
    agg = jnp.concatenate(
        [aggl_ref[0] + aggl_ref[1], aggr_ref[0] + aggr_ref[1]], axis=1)
    h = x_ref[...] + agg
    h = jnp.maximum(
        jnp.dot(h, w1a_ref[...], preferred_element_type=jnp.float32)
        + b1a_ref[...], 0.0)
    h = jnp.dot(h, w1b_ref[...], preferred_element_type=jnp.float32) \
        + b1b_ref[...]
    o_ref[...] = jnp.maximum(h, 0.0)


def _mlp1(x, aggl, aggr, W1a, b1a, W1b, b1b):
    N = x.shape[0]
    BN = 2000
    grid = N // BN
    aggspec = pl.BlockSpec((NC, BN, W), lambda i: (0, i, 0))
    return pl.pallas_call(
        _mlp1_kernel,
        grid=(grid,),
        in_specs=[
            pl.BlockSpec((BN, 128), lambda i: (i, 0)),
            aggspec,
            aggspec,
            pl.BlockSpec((128, 64), lambda i: (0, 0)),
            pl.BlockSpec((1, 64), lambda i: (0, 0)),
            pl.BlockSpec((64, 64), lambda i: (0, 0)),
            pl.BlockSpec((1, 64), lambda i: (0, 0)),
        ],
        out_specs=pl.BlockSpec((BN, 64), lambda i: (i, 0)),
        out_shape=jax.ShapeDtypeStruct((N, 64), jnp.float32),
    )(x, aggl, aggr, W1a, b1a, W1b, b1b)


def _mlp2_pool_kernel(x1_ref, agg_ref, batch_ref, w2a_ref, b2a_ref, w2b_ref,
                      b2b_ref, o_ref, cnt_ref):
    i = pl.program_id(0)

    @pl.when(i == 0)
    def _():
        o_ref[...] = jnp.zeros_like(o_ref)
        cnt_ref[...] = jnp.zeros_like(cnt_ref)

    h = x1_ref[...] + agg_ref[0] + agg_ref[1]
    h = jnp.maximum(
        jnp.dot(h, w2a_ref[...], preferred_element_type=jnp.float32)
        + b2a_ref[...], 0.0)
    t = jnp.dot(h, w2b_ref[...], preferred_element_type=jnp.float32) \
        + b2b_ref[...]
    b = batch_ref[0, 0, :]
    onehot = (b[:, None] == lax.broadcasted_iota(jnp.int32, (1, N_GRAPHS), 1)
              ).astype(jnp.float32)
    o_ref[...] += lax.dot_general(onehot, t, (((0,), (0,)), ((), ())),
                                  preferred_element_type=jnp.float32)
    cnt_ref[...] += jnp.broadcast_to(
        jnp.sum(onehot, axis=0)[:, None], cnt_ref.shape)

    @pl.when(i == pl.num_programs(0) - 1)
    def _():
        o_ref[...] = o_ref[...] / jnp.maximum(cnt_ref[...], 1.0)


def _mlp2_pool(x1, agg, batch3d, W2a, b2a, W2b, b2b):
    N = x1.shape[0]
    BN = 2000
    grid = N // BN
    return pl.pallas_call(
        _mlp2_pool_kernel,
        grid=(grid,),
        in_specs=[
            pl.BlockSpec((BN, 64), lambda i: (i, 0)),
            pl.BlockSpec((NC, BN, 64), lambda i: (0, i, 0)),
            pl.BlockSpec((1, 1, BN), lambda i: (i, 0, 0)),
            pl.BlockSpec((64, 128), lambda i: (0, 0)),
            pl.BlockSpec((1, 128), lambda i: (0, 0)),
            pl.BlockSpec((128, 128), lambda i: (0, 0)),
            pl.BlockSpec((1, 128), lambda i: (0, 0)),
        ],
        out_specs=pl.BlockSpec((N_GRAPHS, 128), lambda i: (0, 0)),
        out_shape=jax.ShapeDtypeStruct((N_GRAPHS, 128), jnp.float32),
        scratch_shapes=[pltpu.VMEM((N_GRAPHS, 128), jnp.float32)],
    )(x1, agg, batch3d, W2a, b2a, W2b, b2b)


def kernel(x, edge_index, edge_attr, batch, We1, be1, W1a, b1a, W1b, b1b,
           We2, be2, W2a, b2a, W2b, b2b):
    E = edge_attr.shape[0]
    N = x.shape[0]
    src = edge_index[0].astype(jnp.int32).reshape(NW, E // (NW * SUB), SUB)
    dst = edge_index[1].astype(jnp.int32).reshape(NW, E // (NW * SUB), SUB)
    e1l, e1r, e2 = _edge_linears(
        edge_attr, We1[:, :W], be1[:W].reshape(1, -1),
        We1[:, W:], be1[W:].reshape(1, -1), We2, be2.reshape(1, -1))
    agg1l = _sc_edge_stage(x[:, :W], src, dst, e1l)
    agg1r = _sc_edge_stage(x[:, W:], src, dst, e1r)
    x1 = _mlp1(x, agg1l, agg1r, W1a, b1a.reshape(1, -1), W1b,
               b1b.reshape(1, -1))
    agg2 = _sc_edge_stage(x1, src, dst, e2)
    batch3d = batch.astype(jnp.int32).reshape(N // 2000, 1, 2000)
    return _mlp2_pool(x1, agg2, batch3d, W2a, b2a.reshape(1, -1),
                      W2b, b2b.reshape(1, -1))


# speedup vs baseline: 3.3932x; 1.0851x over previous
import functools

import jax
import jax.numpy as jnp
from jax import lax
from jax.experimental import pallas as pl
from jax.experimental.pallas import tpu as pltpu
from jax.experimental.pallas import tpu_sc as plsc

N_GRAPHS = 64

NC = 2
NS = 16
NW = NC * NS

SUB = 50
ROWS_PER_CHUNK = 8
C = ROWS_PER_CHUNK * SUB
NBUF = 3

W = 64

AGG_N = 10240
ROWS_PT = AGG_N // NS


def _sc_edge_stage_body(n_chunks, x_hbm, src_hbm, dst_hbm, e_hbm, out_hbm,
                        b0, b1, b2, si0, si1, si2, di0, di1, di2, agg,
                        es0, es1, es2, ss0, ss1, ss2,
                        g0, g1, g2, g3, g4, g5, g6, g7):
    RPC = ROWS_PER_CHUNK
    bufs = [b0, b1, b2]
    sis = [si0, si1, si2]
    dis = [di0, di1, di2]
    esems = [es0, es1, es2]
    ssems = [ss0, ss1, ss2]
    gsems = [g0, g1, g2, g3, g4, g5, g6, g7]
    triples = (n_chunks - 1) // 3

    c = lax.axis_index("c")
    s = lax.axis_index("s")
    wid = c * NS + s
    base = s * ROWS_PT

    def _zrow(r, _):
        for cc in range(W // 16):
            b0[r, pl.ds(cc * 16, 16)] = jnp.zeros((16,), jnp.float32)
        return 0

    lax.fori_loop(0, C, _zrow, 0)
    pltpu.sync_copy(b0, agg.at[pl.ds(base, C)])
    pltpu.sync_copy(b0.at[pl.ds(0, ROWS_PT - C)],
                    agg.at[pl.ds(base + C, ROWS_PT - C)])
    plsc.subcore_barrier()

    def fire_loads(k, p):
        pltpu.async_copy(src_hbm.at[wid, pl.ds(k * RPC, RPC)], sis[p],
                         esems[p])
        pltpu.async_copy(dst_hbm.at[wid, pl.ds(k * RPC, RPC)], dis[p],
                         esems[p])
        pltpu.async_copy(e_hbm.at[pl.ds((wid * n_chunks + k) * C, C)],
                         bufs[p], esems[p])

    def wait_loads(p):
        pltpu.make_async_copy(src_hbm.at[0, pl.ds(0, RPC)], sis[p],
                              esems[p]).wait()
        pltpu.make_async_copy(dst_hbm.at[0, pl.ds(0, RPC)], dis[p],
                              esems[p]).wait()
        pltpu.make_async_copy(e_hbm.at[pl.ds(0, C)], bufs[p],
                              esems[p]).wait()

    def drain_scatters(p):
        for j in range(RPC):
            pltpu.make_async_copy(bufs[p].at[pl.ds(j * SUB, SUB)],
                                  agg.at[dis[p].at[j]], ssems[p]).wait()

    def proc(p):
        wait_loads(p)
        descs = [
            pltpu.async_copy(x_hbm.at[sis[p].at[j]],
                             bufs[p].at[pl.ds(j * SUB, SUB)], gsems[j],
                             add=True)
            for j in range(RPC)
        ]
        for j in range(RPC):
            descs[j].wait()

            def _rrow(r, _):
                for cc in range(W // 16):
                    v = bufs[p][j * SUB + r, pl.ds(cc * 16, 16)]
                    bufs[p][j * SUB + r, pl.ds(cc * 16, 16)] = \
                        jnp.maximum(v, 0.0)
                return 0

            lax.fori_loop(0, SUB, _rrow, 0)
        for j in range(RPC):
            pltpu.async_copy(bufs[p].at[pl.ds(j * SUB, SUB)],
                             agg.at[dis[p].at[j]], ssems[p], add=True)

    fire_loads(0, 0)
    fire_loads(1, 1)

    def _triple(t, _):
        k0 = 3 * t
        proc(0)

        @pl.when(t > 0)
        def _():
            drain_scatters(2)

        fire_loads(k0 + 2, 2)
        proc(1)
        drain_scatters(0)
        fire_loads(k0 + 3, 0)
        proc(2)
        drain_scatters(1)

        @pl.when(t < triples - 1)
        def _():
            fire_loads(k0 + 4, 1)

        return 0

    lax.fori_loop(0, triples, _triple, 0)
    proc(0)
    drain_scatters(2)
    drain_scatters(0)
    plsc.subcore_barrier()
    pltpu.sync_copy(agg.at[pl.ds(base, ROWS_PT)],
                    out_hbm.at[c, pl.ds(base, ROWS_PT)])


def _sc_edge_stage(x, src3d, dst3d, e):
    E = e.shape[0]
    n_chunks = E // (NW * C)
    assert E == NW * C * n_chunks and n_chunks % 3 == 1
    mesh = plsc.VectorSubcoreMesh(core_axis_name="c", subcore_axis_name="s")
    fn = pl.kernel(
        functools.partial(_sc_edge_stage_body, n_chunks),
        out_type=jax.ShapeDtypeStruct((NC, AGG_N, W), jnp.float32),
        mesh=mesh,
        compiler_params=pltpu.CompilerParams(use_tc_tiling_on_sc=False),
        scratch_types=(
            [pltpu.VMEM((C, W), jnp.float32) for _ in range(NBUF)]
            + [pltpu.VMEM((ROWS_PER_CHUNK, SUB), jnp.int32)
               for _ in range(2 * NBUF)]
            + [pltpu.VMEM_SHARED((AGG_N, W), jnp.float32)]
            + [pltpu.SemaphoreType.DMA for _ in range(2 * NBUF + 8)]
        ),
    )
    return fn(x, src3d, dst3d, e)


def _edge_lin_kernel(ea_ref, w1l_ref, b1l_ref, w1r_ref, b1r_ref, w2_ref,
                     b2_ref, o1l_ref, o1r_ref, o2_ref):
    ea = ea_ref[...]
    o1l_ref[...] = jnp.dot(ea, w1l_ref[...],
                           preferred_element_type=jnp.float32) + b1l_ref[...]
    o1r_ref[...] = jnp.dot(ea, w1r_ref[...],
                           preferred_element_type=jnp.float32) + b1r_ref[...]
    o2_ref[...] = jnp.dot(ea, w2_ref[...],
                          preferred_element_type=jnp.float32) + b2_ref[...]


def _edge_linears(edge_attr, W1l, b1l, W1r, b1r, We2, be2):
    E, DE = edge_attr.shape
    BE = 2000
    grid = E // BE
    wspec = pl.BlockSpec((DE, W), lambda i: (0, 0))
    bspec = pl.BlockSpec((1, W), lambda i: (0, 0))
    ospec = pl.BlockSpec((BE, W), lambda i: (i, 0))
    oshape = jax.ShapeDtypeStruct((E, W), jnp.float32)
    return pl.pallas_call(
        _edge_lin_kernel,
        grid=(grid,),
        in_specs=[pl.BlockSpec((BE, DE), lambda i: (i, 0)),
                  wspec, bspec, wspec, bspec, wspec, bspec],
        out_specs=[ospec, ospec, ospec],
        out_shape=[oshape, oshape, oshape],
    )(edge_attr, W1l, b1l, W1r, b1r, We2, be2)


def _mlp1_kernel(x_ref, aggl_ref, aggr_ref, w1a_ref, b1a_ref, w1b_ref,
                 b1b_ref, o_ref):
    agg = jnp.concatenate(
        [aggl_ref[0] + aggl_ref[1], aggr_ref[0] + aggr_ref[1]], axis=1)
    h = x_ref[...] + agg
    h = jnp.maximum(
        jnp.dot(h, w1a_ref[...], preferred_element_type=jnp.float32)
        + b1a_ref[...], 0.0)
    h = jnp.dot(h, w1b_ref[...], preferred_element_type=jnp.float32) \
        + b1b_ref[...]
    o_ref[...] = jnp.maximum(h, 0.0)


def _mlp1(x, aggl, aggr, W1a, b1a, W1b, b1b):
    N = x.shape[0]
    BN = 2000
    grid = N // BN
    aggspec = pl.BlockSpec((NC, BN, W), lambda i: (0, i, 0))
    return pl.pallas_call(
        _mlp1_kernel,
        grid=(grid,),
        in_specs=[
            pl.BlockSpec((BN, 128), lambda i: (i, 0)),
            aggspec,
            aggspec,
            pl.BlockSpec((128, 64), lambda i: (0, 0)),
            pl.BlockSpec((1, 64), lambda i: (0, 0)),
            pl.BlockSpec((64, 64), lambda i: (0, 0)),
            pl.BlockSpec((1, 64), lambda i: (0, 0)),
        ],
        out_specs=pl.BlockSpec((BN, 64), lambda i: (i, 0)),
        out_shape=jax.ShapeDtypeStruct((N, 64), jnp.float32),
    )(x, aggl, aggr, W1a, b1a, W1b, b1b)


def _mlp2_pool_kernel(x1_ref, agg_ref, batch_ref, w2a_ref, b2a_ref, w2b_ref,
                      b2b_ref, o_ref, cnt_ref):
    i = pl.program_id(0)

    @pl.when(i == 0)
    def _():
        o_ref[...] = jnp.zeros_like(o_ref)
        cnt_ref[...] = jnp.zeros_like(cnt_ref)

    h = x1_ref[...] + agg_ref[0] + agg_ref[1]
    h = jnp.maximum(
        jnp.dot(h, w2a_ref[...], preferred_element_type=jnp.float32)
        + b2a_ref[...], 0.0)
    t = jnp.dot(h, w2b_ref[...], preferred_element_type=jnp.float32) \
        + b2b_ref[...]
    b = batch_ref[0, 0, :]
    onehot = (b[:, None] == lax.broadcasted_iota(jnp.int32, (1, N_GRAPHS), 1)
              ).astype(jnp.float32)
    o_ref[...] += lax.dot_general(onehot, t, (((0,), (0,)), ((), ())),
                                  preferred_element_type=jnp.float32)
    cnt_ref[...] += jnp.broadcast_to(
        jnp.sum(onehot, axis=0)[:, None], cnt_ref.shape)

    @pl.when(i == pl.num_programs(0) - 1)
    def _():
        o_ref[...] = o_ref[...] / jnp.maximum(cnt_ref[...], 1.0)


def _mlp2_pool(x1, agg, batch3d, W2a, b2a, W2b, b2b):
    N = x1.shape[0]
    BN = 2000
    grid = N // BN
    return pl.pallas_call(
        _mlp2_pool_kernel,
        grid=(grid,),
        in_specs=[
            pl.BlockSpec((BN, 64), lambda i: (i, 0)),
            pl.BlockSpec((NC, BN, 64), lambda i: (0, i, 0)),
            pl.BlockSpec((1, 1, BN), lambda i: (i, 0, 0)),
            pl.BlockSpec((64, 128), lambda i: (0, 0)),
            pl.BlockSpec((1, 128), lambda i: (0, 0)),
            pl.BlockSpec((128, 128), lambda i: (0, 0)),
            pl.BlockSpec((1, 128), lambda i: (0, 0)),
        ],
        out_specs=pl.BlockSpec((N_GRAPHS, 128), lambda i: (0, 0)),
        out_shape=jax.ShapeDtypeStruct((N_GRAPHS, 128), jnp.float32),
        scratch_shapes=[pltpu.VMEM((N_GRAPHS, 128), jnp.float32)],
    )(x1, agg, batch3d, W2a, b2a, W2b, b2b)


def kernel(x, edge_index, edge_attr, batch, We1, be1, W1a, b1a, W1b, b1b,
           We2, be2, W2a, b2a, W2b, b2b):
    E = edge_attr.shape[0]
    N = x.shape[0]
    src = edge_index[0].astype(jnp.int32).reshape(NW, E // (NW * SUB), SUB)
    dst = edge_index[1].astype(jnp.int32).reshape(NW, E // (NW * SUB), SUB)
    e1l, e1r, e2 = _edge_linears(
        edge_attr, We1[:, :W], be1[:W].reshape(1, -1),
        We1[:, W:], be1[W:].reshape(1, -1), We2, be2.reshape(1, -1))
    agg1l = _sc_edge_stage(x[:, :W], src, dst, e1l)
    agg1r = _sc_edge_stage(x[:, W:], src, dst, e1r)
    x1 = _mlp1(x, agg1l, agg1r, W1a, b1a.reshape(1, -1), W1b,
               b1b.reshape(1, -1))
    agg2 = _sc_edge_stage(x1, src, dst, e2)
    batch3d = batch.astype(jnp.int32).reshape(N // 2000, 1, 2000)
    return _mlp2_pool(x1, agg2, batch3d, W2a, b2a.reshape(1, -1),
                      W2b, b2b.reshape(1, -1))


# speedup vs baseline: 4.1602x; 1.2260x over previous
import functools

import jax
import jax.numpy as jnp
from jax import lax
from jax.experimental import pallas as pl
from jax.experimental.pallas import tpu as pltpu
from jax.experimental.pallas import tpu_sc as plsc

N_GRAPHS = 64

NC = 2
NS = 16
NW = NC * NS

SUB = 50
ROWS_PER_CHUNK = 8
C = ROWS_PER_CHUNK * SUB
NBUF = 3

W = 64

AGG_N = 10240
ROWS_PT = AGG_N // NS


def _sc_edge_stage_body(n_chunks, x_hbm, src_hbm, dst_hbm, e_hbm, out_hbm,
                        b0, b1, b2, si0, si1, si2, di0, di1, di2, agg,
                        es0, es1, es2, ss0, ss1, ss2,
                        g0, g1, g2, g3, g4, g5, g6, g7):
    RPC = ROWS_PER_CHUNK
    bufs = [b0, b1, b2]
    sis = [si0, si1, si2]
    dis = [di0, di1, di2]
    esems = [es0, es1, es2]
    ssems = [ss0, ss1, ss2]
    gsems = [g0, g1, g2, g3, g4, g5, g6, g7]
    triples = (n_chunks - 1) // 3

    c = lax.axis_index("c")
    s = lax.axis_index("s")
    wid = c * NS + s
    base = s * ROWS_PT

    def _zrow(r, _):
        for cc in range(W // 16):
            b0[r, pl.ds(cc * 16, 16)] = jnp.zeros((16,), jnp.float32)
        return 0

    lax.fori_loop(0, C, _zrow, 0)
    pltpu.sync_copy(b0, agg.at[pl.ds(base, C)])
    pltpu.sync_copy(b0.at[pl.ds(0, ROWS_PT - C)],
                    agg.at[pl.ds(base + C, ROWS_PT - C)])
    plsc.subcore_barrier()

    def fire_loads(k, p):
        pltpu.async_copy(src_hbm.at[wid, pl.ds(k * RPC, RPC)], sis[p],
                         esems[p])
        pltpu.async_copy(dst_hbm.at[wid, pl.ds(k * RPC, RPC)], dis[p],
                         esems[p])
        pltpu.async_copy(e_hbm.at[pl.ds((wid * n_chunks + k) * C, C)],
                         bufs[p], esems[p])

    def wait_loads(p):
        pltpu.make_async_copy(src_hbm.at[0, pl.ds(0, RPC)], sis[p],
                              esems[p]).wait()
        pltpu.make_async_copy(dst_hbm.at[0, pl.ds(0, RPC)], dis[p],
                              esems[p]).wait()
        pltpu.make_async_copy(e_hbm.at[pl.ds(0, C)], bufs[p],
                              esems[p]).wait()

    def drain_scatters(p):
        for j in range(RPC):
            pltpu.make_async_copy(bufs[p].at[pl.ds(j * SUB, SUB)],
                                  agg.at[dis[p].at[j]], ssems[p]).wait()

    def proc(p):
        wait_loads(p)
        descs = [
            pltpu.async_copy(x_hbm.at[sis[p].at[j]],
                             bufs[p].at[pl.ds(j * SUB, SUB)], gsems[j],
                             add=True)
            for j in range(RPC)
        ]
        for j in range(RPC):
            descs[j].wait()

            def _rrow(r, _):
                for cc in range(W // 16):
                    v = bufs[p][j * SUB + r, pl.ds(cc * 16, 16)]
                    bufs[p][j * SUB + r, pl.ds(cc * 16, 16)] = \
                        jnp.maximum(v, 0.0)
                return 0

            lax.fori_loop(0, SUB, _rrow, 0)
        for j in range(RPC):
            pltpu.async_copy(bufs[p].at[pl.ds(j * SUB, SUB)],
                             agg.at[dis[p].at[j]], ssems[p], add=True)

    fire_loads(0, 0)
    fire_loads(1, 1)

    def _triple(t, _):
        k0 = 3 * t
        proc(0)

        @pl.when(t > 0)
        def _():
            drain_scatters(2)

        fire_loads(k0 + 2, 2)
        proc(1)
        drain_scatters(0)
        fire_loads(k0 + 3, 0)
        proc(2)
        drain_scatters(1)

        @pl.when(t < triples - 1)
        def _():
            fire_loads(k0 + 4, 1)

        return 0

    lax.fori_loop(0, triples, _triple, 0)
    proc(0)
    drain_scatters(2)
    drain_scatters(0)
    plsc.subcore_barrier()
    pltpu.sync_copy(agg.at[pl.ds(base, ROWS_PT)],
                    out_hbm.at[c, pl.ds(base, ROWS_PT)])


def _sc_edge_stage(x, src3d, dst3d, e):
    E = e.shape[0]
    n_chunks = E // (NW * C)
    assert E == NW * C * n_chunks and n_chunks % 3 == 1
    mesh = plsc.VectorSubcoreMesh(core_axis_name="c", subcore_axis_name="s")
    fn = pl.kernel(
        functools.partial(_sc_edge_stage_body, n_chunks),
        out_type=jax.ShapeDtypeStruct((NC, AGG_N, W), jnp.float32),
        mesh=mesh,
        compiler_params=pltpu.CompilerParams(use_tc_tiling_on_sc=False),
        scratch_types=(
            [pltpu.VMEM((C, W), jnp.float32) for _ in range(NBUF)]
            + [pltpu.VMEM((ROWS_PER_CHUNK, SUB), jnp.int32)
               for _ in range(2 * NBUF)]
            + [pltpu.VMEM_SHARED((AGG_N, W), jnp.float32)]
            + [pltpu.SemaphoreType.DMA for _ in range(2 * NBUF + 8)]
        ),
    )
    return fn(x, src3d, dst3d, e)


def _edge_lin_kernel(ea_ref, wl_ref, bl_ref, wr_ref, br_ref, w2_ref,
                     b2_ref, ol_ref, or_ref, o2_ref):
    ea = ea_ref[...]
    ol_ref[...] = jnp.dot(ea, wl_ref[...],
                          preferred_element_type=jnp.float32) + bl_ref[...]
    or_ref[...] = jnp.dot(ea, wr_ref[...],
                          preferred_element_type=jnp.float32) + br_ref[...]
    o2_ref[...] = jnp.dot(ea, w2_ref[...],
                          preferred_element_type=jnp.float32) + b2_ref[...]


def _edge_linears(ea8, WL, bL, WR, bR, W2k, b2k):
    E8, K = ea8.shape
    KW = 8 * W
    BEK = 2000
    grid = E8 // BEK
    wspec = pl.BlockSpec((K, KW), lambda i: (0, 0))
    bspec = pl.BlockSpec((1, KW), lambda i: (0, 0))
    ospec = pl.BlockSpec((BEK, KW), lambda i: (i, 0))
    oshape = jax.ShapeDtypeStruct((E8, KW), jnp.float32)
    return pl.pallas_call(
        _edge_lin_kernel,
        grid=(grid,),
        in_specs=[pl.BlockSpec((BEK, K), lambda i: (i, 0)),
                  wspec, bspec, wspec, bspec, wspec, bspec],
        out_specs=[ospec, ospec, ospec],
        out_shape=[oshape, oshape, oshape],
    )(ea8, WL, bL, WR, bR, W2k, b2k)


def _mlp1_kernel(x_ref, aggl_ref, aggr_ref, w1a_ref, b1a_ref, w1b_ref,
                 b1b_ref, o_ref):
    agg = jnp.concatenate(
        [aggl_ref[0] + aggl_ref[1], aggr_ref[0] + aggr_ref[1]], axis=1)
    h = x_ref[...] + agg
    h = jnp.maximum(
        jnp.dot(h, w1a_ref[...], preferred_element_type=jnp.float32)
        + b1a_ref[...], 0.0)
    h = jnp.dot(h, w1b_ref[...], preferred_element_type=jnp.float32) \
        + b1b_ref[...]
    o_ref[...] = jnp.maximum(h, 0.0)


def _mlp1(x, aggl, aggr, W1a, b1a, W1b, b1b):
    N = x.shape[0]
    BN = 2000
    grid = N // BN
    aggspec = pl.BlockSpec((NC, BN, W), lambda i: (0, i, 0))
    return pl.pallas_call(
        _mlp1_kernel,
        grid=(grid,),
        in_specs=[
            pl.BlockSpec((BN, 128), lambda i: (i, 0)),
            aggspec,
            aggspec,
            pl.BlockSpec((128, 64), lambda i: (0, 0)),
            pl.BlockSpec((1, 64), lambda i: (0, 0)),
            pl.BlockSpec((64, 64), lambda i: (0, 0)),
            pl.BlockSpec((1, 64), lambda i: (0, 0)),
        ],
        out_specs=pl.BlockSpec((BN, 64), lambda i: (i, 0)),
        out_shape=jax.ShapeDtypeStruct((N, 64), jnp.float32),
    )(x, aggl, aggr, W1a, b1a, W1b, b1b)


def _mlp2_pool_kernel(x1_ref, agg_ref, batch_ref, w2a_ref, b2a_ref, w2b_ref,
                      b2b_ref, o_ref, cnt_ref):
    i = pl.program_id(0)

    @pl.when(i == 0)
    def _():
        o_ref[...] = jnp.zeros_like(o_ref)
        cnt_ref[...] = jnp.zeros_like(cnt_ref)

    h = x1_ref[...] + agg_ref[0] + agg_ref[1]
    h = jnp.maximum(
        jnp.dot(h, w2a_ref[...], preferred_element_type=jnp.float32)
        + b2a_ref[...], 0.0)
    t = jnp.dot(h, w2b_ref[...], preferred_element_type=jnp.float32) \
        + b2b_ref[...]
    b = batch_ref[0, 0, :]
    onehot = (b[:, None] == lax.broadcasted_iota(jnp.int32, (1, N_GRAPHS), 1)
              ).astype(jnp.float32)
    o_ref[...] += lax.dot_general(onehot, t, (((0,), (0,)), ((), ())),
                                  preferred_element_type=jnp.float32)
    cnt_ref[...] += jnp.broadcast_to(
        jnp.sum(onehot, axis=0)[:, None], cnt_ref.shape)

    @pl.when(i == pl.num_programs(0) - 1)
    def _():
        o_ref[...] = o_ref[...] / jnp.maximum(cnt_ref[...], 1.0)


def _mlp2_pool(x1, agg, batch3d, W2a, b2a, W2b, b2b):
    N = x1.shape[0]
    BN = 2000
    grid = N // BN
    return pl.pallas_call(
        _mlp2_pool_kernel,
        grid=(grid,),
        in_specs=[
            pl.BlockSpec((BN, 64), lambda i: (i, 0)),
            pl.BlockSpec((NC, BN, 64), lambda i: (0, i, 0)),
            pl.BlockSpec((1, 1, BN), lambda i: (i, 0, 0)),
            pl.BlockSpec((64, 128), lambda i: (0, 0)),
            pl.BlockSpec((1, 128), lambda i: (0, 0)),
            pl.BlockSpec((128, 128), lambda i: (0, 0)),
            pl.BlockSpec((1, 128), lambda i: (0, 0)),
        ],
        out_specs=pl.BlockSpec((N_GRAPHS, 128), lambda i: (0, 0)),
        out_shape=jax.ShapeDtypeStruct((N_GRAPHS, 128), jnp.float32),
        scratch_shapes=[pltpu.VMEM((N_GRAPHS, 128), jnp.float32)],
    )(x1, agg, batch3d, W2a, b2a, W2b, b2b)


def kernel(x, edge_index, edge_attr, batch, We1, be1, W1a, b1a, W1b, b1b,
           We2, be2, W2a, b2a, W2b, b2b):
    E = edge_attr.shape[0]
    N = x.shape[0]
    DE = edge_attr.shape[1]
    src = edge_index[0].astype(jnp.int32).reshape(NW, E // (NW * SUB), SUB)
    dst = edge_index[1].astype(jnp.int32).reshape(NW, E // (NW * SUB), SUB)
    eye8 = jnp.eye(8, dtype=jnp.float32)
    ea8 = edge_attr.reshape(E // 8, 8 * DE)
    eL8, eR8, e28 = _edge_linears(
        ea8,
        jnp.kron(eye8, We1[:, :W]), jnp.tile(be1[:W], 8).reshape(1, -1),
        jnp.kron(eye8, We1[:, W:]), jnp.tile(be1[W:], 8).reshape(1, -1),
        jnp.kron(eye8, We2), jnp.tile(be2, 8).reshape(1, -1))
    e1l = eL8.reshape(E, W)
    e1r = eR8.reshape(E, W)
    e2 = e28.reshape(E, W)
    agg1l = _sc_edge_stage(x[:, :W], src, dst, e1l)
    agg1r = _sc_edge_stage(x[:, W:], src, dst, e1r)
    x1 = _mlp1(x, agg1l, agg1r, W1a, b1a.reshape(1, -1), W1b,
               b1b.reshape(1, -1))
    agg2 = _sc_edge_stage(x1, src, dst, e2)
    batch3d = batch.astype(jnp.int32).reshape(N // 2000, 1, 2000)
    return _mlp2_pool(x1, agg2, batch3d, W2a, b2a.reshape(1, -1),
                      W2b, b2b.reshape(1, -1))


# speedup vs baseline: 4.2453x; 1.0205x over previous
import functools

import jax
import jax.numpy as jnp
from jax import lax
from jax.experimental import pallas as pl
from jax.experimental.pallas import tpu as pltpu
from jax.experimental.pallas import tpu_sc as plsc

N_GRAPHS = 64

NC = 2
NS = 16
NW = NC * NS

SUB = 50
ROWS_PER_CHUNK = 8
C = ROWS_PER_CHUNK * SUB
NBUF = 3

W = 64

AGG_N = 10240
ROWS_PT = AGG_N // NS


def _sc_edge_stage_body(mode, n_chunks, x_hbm, src_hbm, dst_hbm, e_hbm,
                        out_hbm, b0, b1, b2, si0, si1, si2, di0, di1, di2,
                        agg, es0, es1, es2, ss0, ss1, ss2,
                        g0, g1, g2, g3, g4, g5, g6, g7):
    RPC = ROWS_PER_CHUNK
    bufs = [b0, b1, b2]
    sis = [si0, si1, si2]
    dis = [di0, di1, di2]
    esems = [es0, es1, es2]
    ssems = [ss0, ss1, ss2]
    gsems = [g0, g1, g2, g3, g4, g5, g6, g7]
    triples = n_chunks // 3
    rem = n_chunks - 3 * triples

    c = lax.axis_index("c")
    s = lax.axis_index("s")
    base = s * ROWS_PT
    if mode == "split":
        wid = c * NS + s
        x_src = x_hbm
        e_src = e_hbm
    else:
        wid = s
        x_src = x_hbm.at[c]
        e_src = e_hbm.at[c]

    def _zrow(r, _):
        for cc in range(W // 16):
            b0[r, pl.ds(cc * 16, 16)] = jnp.zeros((16,), jnp.float32)
        return 0

    lax.fori_loop(0, C, _zrow, 0)
    pltpu.sync_copy(b0, agg.at[pl.ds(base, C)])
    pltpu.sync_copy(b0.at[pl.ds(0, ROWS_PT - C)],
                    agg.at[pl.ds(base + C, ROWS_PT - C)])
    plsc.subcore_barrier()

    def fire_loads(k, p):
        pltpu.async_copy(src_hbm.at[wid, pl.ds(k * RPC, RPC)], sis[p],
                         esems[p])
        pltpu.async_copy(dst_hbm.at[wid, pl.ds(k * RPC, RPC)], dis[p],
                         esems[p])
        pltpu.async_copy(e_src.at[pl.ds((wid * n_chunks + k) * C, C)],
                         bufs[p], esems[p])

    def wait_loads(p):
        pltpu.make_async_copy(src_hbm.at[0, pl.ds(0, RPC)], sis[p],
                              esems[p]).wait()
        pltpu.make_async_copy(dst_hbm.at[0, pl.ds(0, RPC)], dis[p],
                              esems[p]).wait()
        pltpu.make_async_copy(e_src.at[pl.ds(0, C)], bufs[p],
                              esems[p]).wait()

    def drain_scatters(p):
        for j in range(RPC):
            pltpu.make_async_copy(bufs[p].at[pl.ds(j * SUB, SUB)],
                                  agg.at[dis[p].at[j]], ssems[p]).wait()

    def proc(p):
        wait_loads(p)
        descs = [
            pltpu.async_copy(x_src.at[sis[p].at[j]],
                             bufs[p].at[pl.ds(j * SUB, SUB)], gsems[j],
                             add=True)
            for j in range(RPC)
        ]
        for j in range(RPC):
            descs[j].wait()

            def _rrow(r, _):
                for cc in range(W // 16):
                    v = bufs[p][j * SUB + r, pl.ds(cc * 16, 16)]
                    bufs[p][j * SUB + r, pl.ds(cc * 16, 16)] = \
                        jnp.maximum(v, 0.0)
                return 0

            lax.fori_loop(0, SUB, _rrow, 0)
        for j in range(RPC):
            pltpu.async_copy(bufs[p].at[pl.ds(j * SUB, SUB)],
                             agg.at[dis[p].at[j]], ssems[p], add=True)

    fire_loads(0, 0)
    fire_loads(1, 1)

    def _triple(t, _):
        k0 = 3 * t
        proc(0)

        @pl.when(t > 0)
        def _():
            drain_scatters(2)

        fire_loads(k0 + 2, 2)
        proc(1)
        drain_scatters(0)
        fire_loads(k0 + 3, 0)
        proc(2)
        drain_scatters(1)

        if rem == 1:
            @pl.when(t < triples - 1)
            def _():
                fire_loads(k0 + 4, 1)
        else:
            fire_loads(k0 + 4, 1)

        return 0

    lax.fori_loop(0, triples, _triple, 0)
    for i in range(rem):
        proc(i)
        drain_scatters((i + 2) % 3)
    drain_scatters(rem - 1)
    plsc.subcore_barrier()
    pltpu.sync_copy(agg.at[pl.ds(base, ROWS_PT)],
                    out_hbm.at[c, pl.ds(base, ROWS_PT)])


def _sc_edge_stage(mode, x, src3d, dst3d, e):
    E = e.shape[-2]
    workers = NW if mode == "split" else NS
    n_chunks = E // (workers * C)
    assert E == workers * C * n_chunks and n_chunks % 3 in (1, 2)
    mesh = plsc.VectorSubcoreMesh(core_axis_name="c", subcore_axis_name="s")
    fn = pl.kernel(
        functools.partial(_sc_edge_stage_body, mode, n_chunks),
        out_type=jax.ShapeDtypeStruct((NC, AGG_N, W), jnp.float32),
        mesh=mesh,
        compiler_params=pltpu.CompilerParams(use_tc_tiling_on_sc=False),
        scratch_types=(
            [pltpu.VMEM((C, W), jnp.float32) for _ in range(NBUF)]
            + [pltpu.VMEM((ROWS_PER_CHUNK, SUB), jnp.int32)
               for _ in range(2 * NBUF)]
            + [pltpu.VMEM_SHARED((AGG_N, W), jnp.float32)]
            + [pltpu.SemaphoreType.DMA for _ in range(2 * NBUF + 8)]
        ),
    )
    return fn(x, src3d, dst3d, e)


def _edge_lin_kernel(ea_ref, wl_ref, bl_ref, wr_ref, br_ref, w2_ref,
                     b2_ref, o1_ref, o2_ref):
    ea = ea_ref[...]
    o1_ref[0] = jnp.dot(ea, wl_ref[...],
                        preferred_element_type=jnp.float32) + bl_ref[...]
    o1_ref[1] = jnp.dot(ea, wr_ref[...],
                        preferred_element_type=jnp.float32) + br_ref[...]
    o2_ref[...] = jnp.dot(ea, w2_ref[...],
                          preferred_element_type=jnp.float32) + b2_ref[...]


def _edge_linears(ea8, WL, bL, WR, bR, W2k, b2k):
    E8, K = ea8.shape
    KW = 8 * W
    BEK = 2000
    grid = E8 // BEK
    wspec = pl.BlockSpec((K, KW), lambda i: (0, 0))
    bspec = pl.BlockSpec((1, KW), lambda i: (0, 0))
    return pl.pallas_call(
        _edge_lin_kernel,
        grid=(grid,),
        in_specs=[pl.BlockSpec((BEK, K), lambda i: (i, 0)),
                  wspec, bspec, wspec, bspec, wspec, bspec],
        out_specs=[pl.BlockSpec((NC, BEK, KW), lambda i: (0, i, 0)),
                   pl.BlockSpec((BEK, KW), lambda i: (i, 0))],
        out_shape=[jax.ShapeDtypeStruct((NC, E8, KW), jnp.float32),
                   jax.ShapeDtypeStruct((E8, KW), jnp.float32)],
    )(ea8, WL, bL, WR, bR, W2k, b2k)


def _mlp1_kernel(x_ref, agg_ref, w1a_ref, b1a_ref, w1b_ref,
                 b1b_ref, o_ref):
    agg = jnp.concatenate([agg_ref[0], agg_ref[1]], axis=1)
    h = x_ref[...] + agg
    h = jnp.maximum(
        jnp.dot(h, w1a_ref[...], preferred_element_type=jnp.float32)
        + b1a_ref[...], 0.0)
    h = jnp.dot(h, w1b_ref[...], preferred_element_type=jnp.float32) \
        + b1b_ref[...]
    o_ref[...] = jnp.maximum(h, 0.0)


def _mlp1(x, agg, W1a, b1a, W1b, b1b):
    N = x.shape[0]
    BN = 2000
    grid = N // BN
    return pl.pallas_call(
        _mlp1_kernel,
        grid=(grid,),
        in_specs=[
            pl.BlockSpec((BN, 128), lambda i: (i, 0)),
            pl.BlockSpec((NC, BN, W), lambda i: (0, i, 0)),
            pl.BlockSpec((128, 64), lambda i: (0, 0)),
            pl.BlockSpec((1, 64), lambda i: (0, 0)),
            pl.BlockSpec((64, 64), lambda i: (0, 0)),
            pl.BlockSpec((1, 64), lambda i: (0, 0)),
        ],
        out_specs=pl.BlockSpec((BN, 64), lambda i: (i, 0)),
        out_shape=jax.ShapeDtypeStruct((N, 64), jnp.float32),
    )(x, agg, W1a, b1a, W1b, b1b)


def _mlp2_pool_kernel(x1_ref, agg_ref, batch_ref, w2a_ref, b2a_ref, w2b_ref,
                      b2b_ref, o_ref, cnt_ref):
    i = pl.program_id(0)

    @pl.when(i == 0)
    def _():
        o_ref[...] = jnp.zeros_like(o_ref)
        cnt_ref[...] = jnp.zeros_like(cnt_ref)

    h = x1_ref[...] + agg_ref[0] + agg_ref[1]
    h = jnp.maximum(
        jnp.dot(h, w2a_ref[...], preferred_element_type=jnp.float32)
        + b2a_ref[...], 0.0)
    t = jnp.dot(h, w2b_ref[...], preferred_element_type=jnp.float32) \
        + b2b_ref[...]
    b = batch_ref[0, 0, :]
    onehot = (b[:, None] == lax.broadcasted_iota(jnp.int32, (1, N_GRAPHS), 1)
              ).astype(jnp.float32)
    o_ref[...] += lax.dot_general(onehot, t, (((0,), (0,)), ((), ())),
                                  preferred_element_type=jnp.float32)
    cnt_ref[...] += jnp.broadcast_to(
        jnp.sum(onehot, axis=0)[:, None], cnt_ref.shape)

    @pl.when(i == pl.num_programs(0) - 1)
    def _():
        o_ref[...] = o_ref[...] / jnp.maximum(cnt_ref[...], 1.0)


def _mlp2_pool(x1, agg, batch3d, W2a, b2a, W2b, b2b):
    N = x1.shape[0]
    BN = 2000
    grid = N // BN
    return pl.pallas_call(
        _mlp2_pool_kernel,
        grid=(grid,),
        in_specs=[
            pl.BlockSpec((BN, 64), lambda i: (i, 0)),
            pl.BlockSpec((NC, BN, 64), lambda i: (0, i, 0)),
            pl.BlockSpec((1, 1, BN), lambda i: (i, 0, 0)),
            pl.BlockSpec((64, 128), lambda i: (0, 0)),
            pl.BlockSpec((1, 128), lambda i: (0, 0)),
            pl.BlockSpec((128, 128), lambda i: (0, 0)),
            pl.BlockSpec((1, 128), lambda i: (0, 0)),
        ],
        out_specs=pl.BlockSpec((N_GRAPHS, 128), lambda i: (0, 0)),
        out_shape=jax.ShapeDtypeStruct((N_GRAPHS, 128), jnp.float32),
        scratch_shapes=[pltpu.VMEM((N_GRAPHS, 128), jnp.float32)],
    )(x1, agg, batch3d, W2a, b2a, W2b, b2b)


def kernel(x, edge_index, edge_attr, batch, We1, be1, W1a, b1a, W1b, b1b,
           We2, be2, W2a, b2a, W2b, b2b):
    E = edge_attr.shape[0]
    N = x.shape[0]
    DE = edge_attr.shape[1]
    srcf = edge_index[0].astype(jnp.int32)
    dstf = edge_index[1].astype(jnp.int32)
    src16 = srcf.reshape(NS, E // (NS * SUB), SUB)
    dst16 = dstf.reshape(NS, E // (NS * SUB), SUB)
    src32 = srcf.reshape(NW, E // (NW * SUB), SUB)
    dst32 = dstf.reshape(NW, E // (NW * SUB), SUB)
    eye8 = jnp.eye(8, dtype=jnp.float32)
    ea8 = edge_attr.reshape(E // 8, 8 * DE)
    e1s8, e28 = _edge_linears(
        ea8,
        jnp.kron(eye8, We1[:, :W]), jnp.tile(be1[:W], 8).reshape(1, -1),
        jnp.kron(eye8, We1[:, W:]), jnp.tile(be1[W:], 8).reshape(1, -1),
        jnp.kron(eye8, We2), jnp.tile(be2, 8).reshape(1, -1))
    e1s = e1s8.reshape(NC, E, W)
    e2 = e28.reshape(E, W)
    x_stack = jnp.stack([x[:, :W], x[:, W:]])
    agg1 = _sc_edge_stage("bycore", x_stack, src16, dst16, e1s)
    x1 = _mlp1(x, agg1, W1a, b1a.reshape(1, -1), W1b,
               b1b.reshape(1, -1))
    agg2 = _sc_edge_stage("split", x1, src32, dst32, e2)
    batch3d = batch.astype(jnp.int32).reshape(N // 2000, 1, 2000)
    return _mlp2_pool(x1, agg2, batch3d, W2a, b2a.reshape(1, -1),
                      W2b, b2b.reshape(1, -1))
